# baseline (speedup 1.0000x reference)
import jax
import jax.numpy as jnp
from jax import lax
from jax.experimental import pallas as pl
from jax.experimental.pallas import tpu as pltpu

NUM_FRAMES = 4096
NUM_CLASSES = 1024
BLOCK_ROWS = 2048
NUM_BLOCKS = NUM_FRAMES // BLOCK_ROWS
STRIPE = 64
LANES = 128
NCHUNK = NUM_CLASSES // LANES
BLANK = 0
NEG = -2147483648


def _stripe_argmax_col(xs):
    mv = xs[:, 0:LANES]
    jv = jnp.zeros((STRIPE, LANES), jnp.float32)
    for c in range(1, NCHUNK):
        xc = xs[:, c * LANES:(c + 1) * LANES]
        gt = xc > mv
        mv = jnp.where(gt, xc, mv)
        jv = jnp.where(gt, jnp.float32(c), jv)
    gm = jnp.max(mv, axis=1, keepdims=True)
    lanef = lax.broadcasted_iota(
        jnp.int32, (STRIPE, LANES), 1).astype(jnp.float32)
    cand = jnp.where(mv == gm, jv * LANES + lanef, jnp.float32(NUM_CLASSES))
    return jnp.min(cand, axis=1, keepdims=True).astype(jnp.int32)


def _decode_block(x_ref, out_ref, carry_ref, idx_ref):
    i = pl.program_id(0)

    @pl.when(i == 0)
    def _init():
        carry_ref[0] = jnp.int32(-1)

    for t in range(BLOCK_ROWS // STRIPE):
        xs = x_ref[pl.ds(t * STRIPE, STRIPE), :]
        idx_ref[pl.ds(t * STRIPE, STRIPE), :] = _stripe_argmax_col(xs)

    idx = idx_ref[...]
    carry = carry_ref[0]
    pos = lax.broadcasted_iota(jnp.int32, (BLOCK_ROWS, 1), 0)
    prev = jnp.where(pos == 0, carry, jnp.roll(idx, 1, axis=0))
    keep = (idx != prev) & (idx != BLANK)
    out_ref[...] = jnp.where(keep, idx, jnp.int32(-1)).reshape(1, BLOCK_ROWS, 1)

    carry_ref[0] = jnp.max(jnp.where(pos == BLOCK_ROWS - 1, idx, NEG))


def kernel(emission):
    out = pl.pallas_call(
        _decode_block,
        grid=(NUM_BLOCKS,),
        in_specs=[
            pl.BlockSpec((BLOCK_ROWS, NUM_CLASSES), lambda i: (i, 0)),
        ],
        out_specs=pl.BlockSpec((1, BLOCK_ROWS, 1), lambda i: (i, 0, 0)),
        out_shape=jax.ShapeDtypeStruct((NUM_BLOCKS, BLOCK_ROWS, 1), jnp.int32),
        scratch_shapes=[
            pltpu.SMEM((1,), jnp.int32),
            pltpu.VMEM((BLOCK_ROWS, 1), jnp.int32),
        ],
    )(emission)
    return out.reshape(NUM_FRAMES)


# speedup vs baseline: 1.5270x; 1.5270x over previous
import jax
import jax.numpy as jnp
from jax import lax
from jax.experimental import pallas as pl
from jax.experimental.pallas import tpu as pltpu

NUM_FRAMES = 4096
NUM_CLASSES = 1024
CH = 512
NCH = NUM_FRAMES // CH
NBUF = 4
AHEAD = 3
BLANK = 0
NEG = -2147483648


def _chunk_argmax(x):
    m = jnp.max(x, axis=1, keepdims=True)
    cls = lax.broadcasted_iota(jnp.int32, x.shape, 1).astype(jnp.float32)
    cand = jnp.where(x == m, cls, jnp.float32(NUM_CLASSES))
    return jnp.min(cand, axis=1).astype(jnp.int32).reshape(1, CH)


def _decode(x_ref, o_ref, b0, b1, b2, b3, s0, s1, s2, s3):
    bufs = (b0, b1, b2, b3)
    sems = (s0, s1, s2, s3)

    def start(k):
        return pltpu.make_async_copy(
            x_ref.at[pl.ds(k * CH, CH), :], bufs[k % NBUF], sems[k % NBUF])

    cps = {}
    for k in range(AHEAD):
        cps[k] = start(k)
        cps[k].start()

    carry = jnp.int32(-1)
    pos = lax.broadcasted_iota(jnp.int32, (1, CH), 1)
    for k in range(NCH):
        if k + AHEAD < NCH:
            cps[k + AHEAD] = start(k + AHEAD)
            cps[k + AHEAD].start()
        cps[k].wait()
        idx = _chunk_argmax(bufs[k % NBUF][...])
        prev = jnp.where(pos == 0, carry, jnp.roll(idx, 1, axis=1))
        keep = (idx != prev) & (idx != BLANK)
        o_ref[pl.ds(0, 1), pl.ds(k * CH, CH)] = jnp.where(
            keep, idx, jnp.int32(-1))
        carry = jnp.max(jnp.where(pos == CH - 1, idx, NEG))


def kernel(emission):
    out = pl.pallas_call(
        _decode,
        in_specs=[pl.BlockSpec(memory_space=pl.ANY)],
        out_specs=pl.BlockSpec(memory_space=pltpu.VMEM),
        out_shape=jax.ShapeDtypeStruct((1, NUM_FRAMES), jnp.int32),
        scratch_shapes=[
            pltpu.VMEM((CH, NUM_CLASSES), jnp.float32),
            pltpu.VMEM((CH, NUM_CLASSES), jnp.float32),
            pltpu.VMEM((CH, NUM_CLASSES), jnp.float32),
            pltpu.VMEM((CH, NUM_CLASSES), jnp.float32),
            pltpu.SemaphoreType.DMA,
            pltpu.SemaphoreType.DMA,
            pltpu.SemaphoreType.DMA,
            pltpu.SemaphoreType.DMA,
        ],
    )(emission)
    return out.reshape(NUM_FRAMES)
